# baseline (speedup 1.0000x reference)
import jax
import jax.numpy as jnp
from jax import lax
from jax.experimental import pallas as pl
from jax.experimental.pallas import tpu as pltpu, tpu_sc as plsc

N = 10000
E = 320000
D = 128
H = 128
C = 10

NC = 2
NS = 16
NW = NC * NS
EPW = E // NW
CH = 80
NCH = EPW // CH
NP = 10240
RPT = NP // NS
R = 2048
G = NP // R


def _sc_mesh():
    return plsc.VectorSubcoreMesh(core_axis_name="c", subcore_axis_name="s")


def _deg_body(edge_hbm, out_hbm, src_v, dst_v, dego_v, degi_v):
    c = lax.axis_index("c")
    s = lax.axis_index("s")
    wid = c * NS + s
    base = pl.multiple_of(wid * EPW, 8)
    zeros16 = jnp.zeros((16,), jnp.float32)
    ones16 = jnp.ones((16,), jnp.float32)

    def zbody(i, carry):
        dego_v[pl.ds(i * 16, 16)] = zeros16
        degi_v[pl.ds(i * 16, 16)] = zeros16
        return carry

    lax.fori_loop(0, N // 16, zbody, 0)

    pltpu.sync_copy(edge_hbm.at[pl.ds(base, EPW)], src_v)
    pltpu.sync_copy(edge_hbm.at[pl.ds(E + base, EPW)], dst_v)

    def abody(i, carry):
        sidx = src_v[pl.ds(i * 16, 16)]
        didx = dst_v[pl.ds(i * 16, 16)]
        plsc.addupdate_scatter(dego_v, [sidx], ones16)
        plsc.addupdate_scatter(degi_v, [didx], ones16)
        return carry

    lax.fori_loop(0, EPW // 16, abody, 0)

    pltpu.sync_copy(dego_v, out_hbm.at[pl.ds(pl.multiple_of(wid * N, 8), N)])
    pltpu.sync_copy(
        degi_v, out_hbm.at[pl.ds(pl.multiple_of((NW + wid) * N, 8), N)])


def _deg_call(edge_flat):
    f = pl.kernel(
        _deg_body,
        out_type=jax.ShapeDtypeStruct((2 * NW * N,), jnp.float32),
        mesh=_sc_mesh(),
        compiler_params=pltpu.CompilerParams(needs_layout_passes=False),
        scratch_types=[
            pltpu.VMEM((EPW,), jnp.int32),
            pltpu.VMEM((EPW,), jnp.int32),
            pltpu.VMEM((N,), jnp.float32),
            pltpu.VMEM((N,), jnp.float32),
        ],
    )
    return f(edge_flat)


def _agg_body(y_hbm, edge_hbm, out_hbm, src_all, dst_all, dstc_v, rows_v,
              acc_sh, sem, sem_s):
    c = lax.axis_index("c")
    s = lax.axis_index("s")
    wid = c * NS + s
    base = pl.multiple_of(wid * EPW, 8)
    zeros16 = jnp.zeros((16,), jnp.float32)

    pltpu.sync_copy(edge_hbm.at[pl.ds(base, EPW)], src_all)
    pltpu.sync_copy(edge_hbm.at[pl.ds(E + base, EPW)], dst_all)

    def zbody(i, carry):
        rows_v[0, i // 8, pl.ds((i % 8) * 16, 16)] = zeros16
        return carry

    lax.fori_loop(0, CH * (D // 16), zbody, 0)
    row0 = pl.multiple_of(s * RPT, 8)
    for r in range(RPT // CH):
        pltpu.sync_copy(rows_v.at[0], acc_sh.at[pl.ds(row0 + r * CH, CH)])
    plsc.subcore_barrier()

    def src_sl(k):
        return src_all.at[pl.ds(pl.multiple_of(k * CH, 8), CH)]

    def start_gather(k, b):
        pltpu.async_copy(y_hbm.at[src_sl(k)], rows_v.at[b], sem.at[b])

    def wait_gather(k, b):
        pltpu.make_async_copy(y_hbm.at[src_sl(k)], rows_v.at[b],
                              sem.at[b]).wait()

    def start_scatter(k, b):
        off = pl.multiple_of(k * CH, 8)
        for i in range(CH // 16):
            dstc_v[b, pl.ds(i * 16, 16)] = dst_all[pl.ds(off + i * 16, 16)]
        pltpu.async_copy(rows_v.at[b], acc_sh.at[dstc_v.at[b]], sem_s.at[b],
                         add=True)

    def wait_scatter(b):
        pltpu.make_async_copy(rows_v.at[b], acc_sh.at[dstc_v.at[b]],
                              sem_s.at[b]).wait()

    start_gather(0, 0)

    def ebody(g, carry):
        k = g * 2
        wait_gather(k, 0)

        @pl.when(k >= 2)
        def _():
            wait_scatter(1)

        start_gather(k + 1, 1)
        start_scatter(k, 0)
        wait_gather(k + 1, 1)
        wait_scatter(0)

        @pl.when(k + 2 < NCH)
        def _():
            start_gather(k + 2, 0)

        start_scatter(k + 1, 1)
        return carry

    lax.fori_loop(0, NCH // 2, ebody, 0)
    wait_scatter(1)
    wait_gather(NCH - 1, 0)
    start_scatter(NCH - 1, 0)
    wait_scatter(0)
    plsc.subcore_barrier()
    pltpu.sync_copy(
        acc_sh.at[pl.ds(row0, RPT)],
        out_hbm.at[pl.ds(pl.multiple_of(c * NP + s * RPT, 8), RPT)])


def _agg_call(y, edge_flat):
    f = pl.kernel(
        _agg_body,
        out_type=jax.ShapeDtypeStruct((NC * NP, D), jnp.float32),
        mesh=_sc_mesh(),
        compiler_params=pltpu.CompilerParams(needs_layout_passes=False),
        scratch_types=[
            pltpu.VMEM((EPW,), jnp.int32),
            pltpu.VMEM((EPW,), jnp.int32),
            pltpu.VMEM((2, CH), jnp.int32),
            pltpu.VMEM((2, CH, D), jnp.float32),
            pltpu.VMEM_SHARED((NP, D), jnp.float32),
            pltpu.SemaphoreType.DMA((2,)),
            pltpu.SemaphoreType.DMA((2,)),
        ],
    )
    return f(y, edge_flat).reshape(NC, NP, D)


def _norms_body(deg_ref, h_ref, norms_ref, y1_ref):
    dp = deg_ref[...]
    dego = jnp.sum(dp[:NW], axis=0)
    degi = jnp.sum(dp[NW:], axis=0)
    no = lax.rsqrt(jnp.maximum(dego, 1.0))
    ni = lax.rsqrt(jnp.maximum(degi, 1.0))
    norms_ref[...] = jnp.stack([no, ni])
    y1_ref[...] = h_ref[...] * no[:, None]


def _norms_call(deg_part, h):
    return pl.pallas_call(
        _norms_body,
        grid=(G,),
        in_specs=[
            pl.BlockSpec((2 * NW, R), lambda i: (0, i)),
            pl.BlockSpec((R, D), lambda i: (i, 0)),
        ],
        out_specs=[
            pl.BlockSpec((2, R), lambda i: (0, i)),
            pl.BlockSpec((R, D), lambda i: (i, 0)),
        ],
        out_shape=[
            jax.ShapeDtypeStruct((2, N), jnp.float32),
            jax.ShapeDtypeStruct((N, D), jnp.float32),
        ],
    )(deg_part.reshape(2 * NW, N), h)


def _layer_body(part_ref, norms_ref, w_ref, b_ref, y2_ref):
    a = part_ref[0] + part_ref[1]
    nr = norms_ref[...]
    a = a * nr[1][:, None]
    z = jnp.dot(a, w_ref[...], preferred_element_type=jnp.float32) + b_ref[...]
    z = jnp.maximum(z, 0.0)
    y2_ref[...] = z * nr[0][:, None]


def _layer_call(part, norms, W, b):
    return pl.pallas_call(
        _layer_body,
        grid=(G,),
        in_specs=[
            pl.BlockSpec((NC, R, D), lambda i: (0, i, 0)),
            pl.BlockSpec((2, R), lambda i: (0, i)),
            pl.BlockSpec((D, H), lambda i: (0, 0)),
            pl.BlockSpec((1, H), lambda i: (0, 0)),
        ],
        out_specs=pl.BlockSpec((R, H), lambda i: (i, 0)),
        out_shape=jax.ShapeDtypeStruct((N, H), jnp.float32),
    )(part, norms, W, b.reshape(1, H))


def _final_body(part_ref, norms_ref, w_ref, b_ref, wc_ref, bc_ref, out_ref,
                acc_ref):
    i = pl.program_id(0)
    a = part_ref[0] + part_ref[1]
    nr = norms_ref[...]
    a = a * nr[1][:, None]
    z = jnp.dot(a, w_ref[...], preferred_element_type=jnp.float32) + b_ref[...]
    z = jnp.maximum(z, 0.0)
    rows = lax.broadcasted_iota(jnp.int32, z.shape, 0) + i * R
    z = jnp.where(rows < N, z, 0.0)

    @pl.when(i == 0)
    def _():
        acc_ref[...] = jnp.zeros_like(acc_ref)

    acc_ref[...] += jnp.sum(z, axis=0, keepdims=True)
    out_ref[...] = (
        jnp.dot(acc_ref[...] * (1.0 / N), wc_ref[...],
                preferred_element_type=jnp.float32) + bc_ref[...]
    )


def _final_call(part, norms, W, b, Wc, bc):
    return pl.pallas_call(
        _final_body,
        grid=(G,),
        in_specs=[
            pl.BlockSpec((NC, R, D), lambda i: (0, i, 0)),
            pl.BlockSpec((2, R), lambda i: (0, i)),
            pl.BlockSpec((H, H), lambda i: (0, 0)),
            pl.BlockSpec((1, H), lambda i: (0, 0)),
            pl.BlockSpec((H, C), lambda i: (0, 0)),
            pl.BlockSpec((1, C), lambda i: (0, 0)),
        ],
        out_specs=pl.BlockSpec((1, C), lambda i: (0, 0)),
        out_shape=jax.ShapeDtypeStruct((1, C), jnp.float32),
        scratch_shapes=[pltpu.VMEM((1, H), jnp.float32)],
    )(part, norms, W, b.reshape(1, H), Wc, bc.reshape(1, C))


def kernel(h, edge_index, W1, b1, W2, b2, Wc, bc):
    edge_flat = edge_index.astype(jnp.int32).reshape(2 * E)
    deg_part = _deg_call(edge_flat)
    norms, y1 = _norms_call(deg_part, h)
    part1 = _agg_call(y1, edge_flat)
    y2 = _layer_call(part1, norms, W1, b1)
    part2 = _agg_call(y2, edge_flat)
    return _final_call(part2, norms, W2, b2, Wc, bc)


# speedup vs baseline: 1.0141x; 1.0141x over previous
import jax
import jax.numpy as jnp
from jax import lax
from jax.experimental import pallas as pl
from jax.experimental.pallas import tpu as pltpu, tpu_sc as plsc

N = 10000
E = 320000
D = 128
H = 128
C = 10

NC = 2
NS = 16
NW = NC * NS
EPW = E // NW
CH = 80
NCH = EPW // CH
NP = 10240
RPT = NP // NS
R = 5120
G = NP // R


def _sc_mesh():
    return plsc.VectorSubcoreMesh(core_axis_name="c", subcore_axis_name="s")


def _deg_body(edge_hbm, out_hbm, src_v, dst_v, dego_v, degi_v):
    c = lax.axis_index("c")
    s = lax.axis_index("s")
    wid = c * NS + s
    base = pl.multiple_of(wid * EPW, 8)
    zeros16 = jnp.zeros((16,), jnp.float32)
    ones16 = jnp.ones((16,), jnp.float32)

    def zbody(i, carry):
        dego_v[pl.ds(i * 16, 16)] = zeros16
        degi_v[pl.ds(i * 16, 16)] = zeros16
        return carry

    lax.fori_loop(0, N // 16, zbody, 0)

    pltpu.sync_copy(edge_hbm.at[pl.ds(base, EPW)], src_v)
    pltpu.sync_copy(edge_hbm.at[pl.ds(E + base, EPW)], dst_v)

    def abody(i, carry):
        sidx = src_v[pl.ds(i * 16, 16)]
        didx = dst_v[pl.ds(i * 16, 16)]
        plsc.addupdate_scatter(dego_v, [sidx], ones16)
        plsc.addupdate_scatter(degi_v, [didx], ones16)
        return carry

    lax.fori_loop(0, EPW // 16, abody, 0)

    pltpu.sync_copy(dego_v, out_hbm.at[pl.ds(pl.multiple_of(wid * N, 8), N)])
    pltpu.sync_copy(
        degi_v, out_hbm.at[pl.ds(pl.multiple_of((NW + wid) * N, 8), N)])


def _deg_call(edge_flat):
    f = pl.kernel(
        _deg_body,
        out_type=jax.ShapeDtypeStruct((2 * NW * N,), jnp.float32),
        mesh=_sc_mesh(),
        compiler_params=pltpu.CompilerParams(needs_layout_passes=False),
        scratch_types=[
            pltpu.VMEM((EPW,), jnp.int32),
            pltpu.VMEM((EPW,), jnp.int32),
            pltpu.VMEM((N,), jnp.float32),
            pltpu.VMEM((N,), jnp.float32),
        ],
    )
    return f(edge_flat)


def _agg_body(y_hbm, edge_hbm, out_hbm, src_all, dst_all, dstc_v, rows_v,
              acc_sh, sem, sem_s):
    c = lax.axis_index("c")
    s = lax.axis_index("s")
    wid = c * NS + s
    base = pl.multiple_of(wid * EPW, 8)
    zeros16 = jnp.zeros((16,), jnp.float32)

    pltpu.sync_copy(edge_hbm.at[pl.ds(base, EPW)], src_all)
    pltpu.sync_copy(edge_hbm.at[pl.ds(E + base, EPW)], dst_all)

    def zbody(i, carry):
        rows_v[0, i // 8, pl.ds((i % 8) * 16, 16)] = zeros16
        return carry

    lax.fori_loop(0, CH * (D // 16), zbody, 0)
    row0 = pl.multiple_of(s * RPT, 8)
    for r in range(RPT // CH):
        pltpu.sync_copy(rows_v.at[0], acc_sh.at[pl.ds(row0 + r * CH, CH)])
    plsc.subcore_barrier()

    def src_sl(k):
        return src_all.at[pl.ds(pl.multiple_of(k * CH, 8), CH)]

    def start_gather(k, b):
        pltpu.async_copy(y_hbm.at[src_sl(k)], rows_v.at[b], sem.at[b])

    def wait_gather(k, b):
        pltpu.make_async_copy(y_hbm.at[src_sl(k)], rows_v.at[b],
                              sem.at[b]).wait()

    def start_scatter(k, b):
        off = pl.multiple_of(k * CH, 8)
        for i in range(CH // 16):
            dstc_v[b, pl.ds(i * 16, 16)] = dst_all[pl.ds(off + i * 16, 16)]
        pltpu.async_copy(rows_v.at[b], acc_sh.at[dstc_v.at[b]], sem_s.at[b],
                         add=True)

    def wait_scatter(b):
        pltpu.make_async_copy(rows_v.at[b], acc_sh.at[dstc_v.at[b]],
                              sem_s.at[b]).wait()

    start_gather(0, 0)

    def ebody(g, carry):
        k = g * 2
        wait_gather(k, 0)

        @pl.when(k >= 2)
        def _():
            wait_scatter(1)

        start_gather(k + 1, 1)
        start_scatter(k, 0)
        wait_gather(k + 1, 1)
        wait_scatter(0)

        @pl.when(k + 2 < NCH)
        def _():
            start_gather(k + 2, 0)

        start_scatter(k + 1, 1)
        return carry

    lax.fori_loop(0, NCH // 2, ebody, 0)
    wait_scatter(1)
    wait_gather(NCH - 1, 0)
    start_scatter(NCH - 1, 0)
    wait_scatter(0)
    plsc.subcore_barrier()
    pltpu.sync_copy(
        acc_sh.at[pl.ds(row0, RPT)],
        out_hbm.at[pl.ds(pl.multiple_of(c * NP + s * RPT, 8), RPT)])


def _agg_call(y, edge_flat):
    f = pl.kernel(
        _agg_body,
        out_type=jax.ShapeDtypeStruct((NC * NP, D), jnp.float32),
        mesh=_sc_mesh(),
        compiler_params=pltpu.CompilerParams(needs_layout_passes=False),
        scratch_types=[
            pltpu.VMEM((EPW,), jnp.int32),
            pltpu.VMEM((EPW,), jnp.int32),
            pltpu.VMEM((2, CH), jnp.int32),
            pltpu.VMEM((2, CH, D), jnp.float32),
            pltpu.VMEM_SHARED((NP, D), jnp.float32),
            pltpu.SemaphoreType.DMA((2,)),
            pltpu.SemaphoreType.DMA((2,)),
        ],
    )
    return f(y, edge_flat).reshape(NC, NP, D)


def _norms_body(deg_ref, h_ref, norms_ref, y1_ref):
    dp = deg_ref[...]
    dego = jnp.sum(dp[:NW], axis=0)
    degi = jnp.sum(dp[NW:], axis=0)
    no = lax.rsqrt(jnp.maximum(dego, 1.0))
    ni = lax.rsqrt(jnp.maximum(degi, 1.0))
    norms_ref[...] = jnp.stack([no, ni])
    y1_ref[...] = h_ref[...] * no[:, None]


def _norms_call(deg_part, h):
    return pl.pallas_call(
        _norms_body,
        grid=(G,),
        in_specs=[
            pl.BlockSpec((2 * NW, R), lambda i: (0, i)),
            pl.BlockSpec((R, D), lambda i: (i, 0)),
        ],
        out_specs=[
            pl.BlockSpec((2, R), lambda i: (0, i)),
            pl.BlockSpec((R, D), lambda i: (i, 0)),
        ],
        out_shape=[
            jax.ShapeDtypeStruct((2, N), jnp.float32),
            jax.ShapeDtypeStruct((N, D), jnp.float32),
        ],
    )(deg_part.reshape(2 * NW, N), h)


def _layer_body(part_ref, norms_ref, w_ref, b_ref, y2_ref):
    a = part_ref[0] + part_ref[1]
    nr = norms_ref[...]
    a = a * nr[1][:, None]
    z = jnp.dot(a, w_ref[...], preferred_element_type=jnp.float32) + b_ref[...]
    z = jnp.maximum(z, 0.0)
    y2_ref[...] = z * nr[0][:, None]


def _layer_call(part, norms, W, b):
    return pl.pallas_call(
        _layer_body,
        grid=(G,),
        in_specs=[
            pl.BlockSpec((NC, R, D), lambda i: (0, i, 0)),
            pl.BlockSpec((2, R), lambda i: (0, i)),
            pl.BlockSpec((D, H), lambda i: (0, 0)),
            pl.BlockSpec((1, H), lambda i: (0, 0)),
        ],
        out_specs=pl.BlockSpec((R, H), lambda i: (i, 0)),
        out_shape=jax.ShapeDtypeStruct((N, H), jnp.float32),
    )(part, norms, W, b.reshape(1, H))


def _final_body(part_ref, norms_ref, w_ref, b_ref, wc_ref, bc_ref, out_ref,
                acc_ref):
    i = pl.program_id(0)
    a = part_ref[0] + part_ref[1]
    nr = norms_ref[...]
    a = a * nr[1][:, None]
    z = jnp.dot(a, w_ref[...], preferred_element_type=jnp.float32) + b_ref[...]
    z = jnp.maximum(z, 0.0)
    rows = lax.broadcasted_iota(jnp.int32, z.shape, 0) + i * R
    z = jnp.where(rows < N, z, 0.0)

    @pl.when(i == 0)
    def _():
        acc_ref[...] = jnp.zeros_like(acc_ref)

    acc_ref[...] += jnp.sum(z, axis=0, keepdims=True)
    out_ref[...] = (
        jnp.dot(acc_ref[...] * (1.0 / N), wc_ref[...],
                preferred_element_type=jnp.float32) + bc_ref[...]
    )


def _final_call(part, norms, W, b, Wc, bc):
    return pl.pallas_call(
        _final_body,
        grid=(G,),
        in_specs=[
            pl.BlockSpec((NC, R, D), lambda i: (0, i, 0)),
            pl.BlockSpec((2, R), lambda i: (0, i)),
            pl.BlockSpec((H, H), lambda i: (0, 0)),
            pl.BlockSpec((1, H), lambda i: (0, 0)),
            pl.BlockSpec((H, C), lambda i: (0, 0)),
            pl.BlockSpec((1, C), lambda i: (0, 0)),
        ],
        out_specs=pl.BlockSpec((1, C), lambda i: (0, 0)),
        out_shape=jax.ShapeDtypeStruct((1, C), jnp.float32),
        scratch_shapes=[pltpu.VMEM((1, H), jnp.float32)],
    )(part, norms, W, b.reshape(1, H), Wc, bc.reshape(1, C))


def kernel(h, edge_index, W1, b1, W2, b2, Wc, bc):
    edge_flat = edge_index.astype(jnp.int32).reshape(2 * E)
    deg_part = _deg_call(edge_flat)
    norms, y1 = _norms_call(deg_part, h)
    part1 = _agg_call(y1, edge_flat)
    y2 = _layer_call(part1, norms, W1, b1)
    part2 = _agg_call(y2, edge_flat)
    return _final_call(part2, norms, W2, b2, Wc, bc)


# speedup vs baseline: 1.0238x; 1.0096x over previous
import jax
import jax.numpy as jnp
from jax import lax
from jax.experimental import pallas as pl
from jax.experimental.pallas import tpu as pltpu, tpu_sc as plsc

N = 10000
E = 320000
D = 128
H = 128
C = 10

NC = 2
NS = 16
NW = NC * NS
EPW = E // NW
CH = 80
NCH = EPW // CH
NP = 10240
RPT = NP // NS
R = 5120
G = NP // R


def _sc_mesh():
    return plsc.VectorSubcoreMesh(core_axis_name="c", subcore_axis_name="s")


def _deg_body(edge_hbm, out_hbm, src_v, dst_v, dego_v, degi_v, sem):
    c = lax.axis_index("c")
    s = lax.axis_index("s")
    wid = c * NS + s
    base = pl.multiple_of(wid * EPW, 8)
    zeros16 = jnp.zeros((16,), jnp.float32)
    ones16 = jnp.ones((16,), jnp.float32)

    pltpu.async_copy(edge_hbm.at[pl.ds(base, EPW)], src_v, sem.at[0])
    pltpu.async_copy(edge_hbm.at[pl.ds(E + base, EPW)], dst_v, sem.at[1])

    def zbody(i, carry):
        for j in range(5):
            dego_v[pl.ds((i * 5 + j) * 16, 16)] = zeros16
            degi_v[pl.ds((i * 5 + j) * 16, 16)] = zeros16
        return carry

    lax.fori_loop(0, N // 80, zbody, 0)

    pltpu.make_async_copy(edge_hbm.at[pl.ds(base, EPW)], src_v,
                          sem.at[0]).wait()
    pltpu.make_async_copy(edge_hbm.at[pl.ds(E + base, EPW)], dst_v,
                          sem.at[1]).wait()

    def abody(i, carry):
        for j in range(5):
            off = (i * 5 + j) * 16
            sidx = src_v[pl.ds(off, 16)]
            didx = dst_v[pl.ds(off, 16)]
            plsc.addupdate_scatter(dego_v, [sidx], ones16)
            plsc.addupdate_scatter(degi_v, [didx], ones16)
        return carry

    lax.fori_loop(0, EPW // 80, abody, 0)

    pltpu.sync_copy(dego_v, out_hbm.at[pl.ds(pl.multiple_of(wid * N, 8), N)])
    pltpu.sync_copy(
        degi_v, out_hbm.at[pl.ds(pl.multiple_of((NW + wid) * N, 8), N)])


def _deg_call(edge_flat):
    f = pl.kernel(
        _deg_body,
        out_type=jax.ShapeDtypeStruct((2 * NW * N,), jnp.float32),
        mesh=_sc_mesh(),
        compiler_params=pltpu.CompilerParams(needs_layout_passes=False),
        scratch_types=[
            pltpu.VMEM((EPW,), jnp.int32),
            pltpu.VMEM((EPW,), jnp.int32),
            pltpu.VMEM((N,), jnp.float32),
            pltpu.VMEM((N,), jnp.float32),
            pltpu.SemaphoreType.DMA((2,)),
        ],
    )
    return f(edge_flat)


def _agg_body(y_hbm, edge_hbm, out_hbm, src_all, dst_all, dstc_v, rows_v,
              acc_sh, sem, sem_s):
    c = lax.axis_index("c")
    s = lax.axis_index("s")
    wid = c * NS + s
    base = pl.multiple_of(wid * EPW, 8)
    zeros16 = jnp.zeros((16,), jnp.float32)

    pltpu.sync_copy(edge_hbm.at[pl.ds(base, EPW)], src_all)
    pltpu.sync_copy(edge_hbm.at[pl.ds(E + base, EPW)], dst_all)

    def zbody(i, carry):
        rows_v[0, i // 8, pl.ds((i % 8) * 16, 16)] = zeros16
        return carry

    lax.fori_loop(0, CH * (D // 16), zbody, 0)
    row0 = pl.multiple_of(s * RPT, 8)
    for r in range(RPT // CH):
        pltpu.sync_copy(rows_v.at[0], acc_sh.at[pl.ds(row0 + r * CH, CH)])
    plsc.subcore_barrier()

    def src_sl(k):
        return src_all.at[pl.ds(pl.multiple_of(k * CH, 8), CH)]

    def start_gather(k, b):
        pltpu.async_copy(y_hbm.at[src_sl(k)], rows_v.at[b], sem.at[b])

    def wait_gather(k, b):
        pltpu.make_async_copy(y_hbm.at[src_sl(k)], rows_v.at[b],
                              sem.at[b]).wait()

    def start_scatter(k, b):
        off = pl.multiple_of(k * CH, 8)
        for i in range(CH // 16):
            dstc_v[b, pl.ds(i * 16, 16)] = dst_all[pl.ds(off + i * 16, 16)]
        pltpu.async_copy(rows_v.at[b], acc_sh.at[dstc_v.at[b]], sem_s.at[b],
                         add=True)

    def wait_scatter(b):
        pltpu.make_async_copy(rows_v.at[b], acc_sh.at[dstc_v.at[b]],
                              sem_s.at[b]).wait()

    start_gather(0, 0)

    def ebody(g, carry):
        k = g * 2
        wait_gather(k, 0)

        @pl.when(k >= 2)
        def _():
            wait_scatter(1)

        start_gather(k + 1, 1)
        start_scatter(k, 0)
        wait_gather(k + 1, 1)
        wait_scatter(0)

        @pl.when(k + 2 < NCH)
        def _():
            start_gather(k + 2, 0)

        start_scatter(k + 1, 1)
        return carry

    lax.fori_loop(0, NCH // 2, ebody, 0)
    wait_scatter(1)
    wait_gather(NCH - 1, 0)
    start_scatter(NCH - 1, 0)
    wait_scatter(0)
    plsc.subcore_barrier()
    pltpu.sync_copy(
        acc_sh.at[pl.ds(row0, RPT)],
        out_hbm.at[pl.ds(pl.multiple_of(c * NP + s * RPT, 8), RPT)])


def _agg_call(y, edge_flat):
    f = pl.kernel(
        _agg_body,
        out_type=jax.ShapeDtypeStruct((NC * NP, D), jnp.float32),
        mesh=_sc_mesh(),
        compiler_params=pltpu.CompilerParams(needs_layout_passes=False),
        scratch_types=[
            pltpu.VMEM((EPW,), jnp.int32),
            pltpu.VMEM((EPW,), jnp.int32),
            pltpu.VMEM((2, CH), jnp.int32),
            pltpu.VMEM((2, CH, D), jnp.float32),
            pltpu.VMEM_SHARED((NP, D), jnp.float32),
            pltpu.SemaphoreType.DMA((2,)),
            pltpu.SemaphoreType.DMA((2,)),
        ],
    )
    return f(y, edge_flat).reshape(NC, NP, D)


def _norms_body(deg_ref, h_ref, norms_ref, y1_ref):
    dp = deg_ref[...]
    dego = jnp.sum(dp[:NW], axis=0)
    degi = jnp.sum(dp[NW:], axis=0)
    no = lax.rsqrt(jnp.maximum(dego, 1.0))
    ni = lax.rsqrt(jnp.maximum(degi, 1.0))
    norms_ref[...] = jnp.stack([no, ni])
    y1_ref[...] = h_ref[...] * no[:, None]


def _norms_call(deg_part, h):
    return pl.pallas_call(
        _norms_body,
        grid=(G,),
        in_specs=[
            pl.BlockSpec((2 * NW, R), lambda i: (0, i)),
            pl.BlockSpec((R, D), lambda i: (i, 0)),
        ],
        out_specs=[
            pl.BlockSpec((2, R), lambda i: (0, i)),
            pl.BlockSpec((R, D), lambda i: (i, 0)),
        ],
        out_shape=[
            jax.ShapeDtypeStruct((2, N), jnp.float32),
            jax.ShapeDtypeStruct((N, D), jnp.float32),
        ],
    )(deg_part.reshape(2 * NW, N), h)


def _layer_body(part_ref, norms_ref, w_ref, b_ref, y2_ref):
    a = part_ref[0] + part_ref[1]
    nr = norms_ref[...]
    a = a * nr[1][:, None]
    z = jnp.dot(a, w_ref[...], preferred_element_type=jnp.float32) + b_ref[...]
    z = jnp.maximum(z, 0.0)
    y2_ref[...] = z * nr[0][:, None]


def _layer_call(part, norms, W, b):
    return pl.pallas_call(
        _layer_body,
        grid=(G,),
        in_specs=[
            pl.BlockSpec((NC, R, D), lambda i: (0, i, 0)),
            pl.BlockSpec((2, R), lambda i: (0, i)),
            pl.BlockSpec((D, H), lambda i: (0, 0)),
            pl.BlockSpec((1, H), lambda i: (0, 0)),
        ],
        out_specs=pl.BlockSpec((R, H), lambda i: (i, 0)),
        out_shape=jax.ShapeDtypeStruct((N, H), jnp.float32),
    )(part, norms, W, b.reshape(1, H))


def _final_body(part_ref, norms_ref, w_ref, b_ref, wc_ref, bc_ref, out_ref,
                acc_ref):
    i = pl.program_id(0)
    a = part_ref[0] + part_ref[1]
    nr = norms_ref[...]
    a = a * nr[1][:, None]
    z = jnp.dot(a, w_ref[...], preferred_element_type=jnp.float32) + b_ref[...]
    z = jnp.maximum(z, 0.0)
    rows = lax.broadcasted_iota(jnp.int32, z.shape, 0) + i * R
    z = jnp.where(rows < N, z, 0.0)

    @pl.when(i == 0)
    def _():
        acc_ref[...] = jnp.zeros_like(acc_ref)

    acc_ref[...] += jnp.sum(z, axis=0, keepdims=True)
    out_ref[...] = (
        jnp.dot(acc_ref[...] * (1.0 / N), wc_ref[...],
                preferred_element_type=jnp.float32) + bc_ref[...]
    )


def _final_call(part, norms, W, b, Wc, bc):
    return pl.pallas_call(
        _final_body,
        grid=(G,),
        in_specs=[
            pl.BlockSpec((NC, R, D), lambda i: (0, i, 0)),
            pl.BlockSpec((2, R), lambda i: (0, i)),
            pl.BlockSpec((H, H), lambda i: (0, 0)),
            pl.BlockSpec((1, H), lambda i: (0, 0)),
            pl.BlockSpec((H, C), lambda i: (0, 0)),
            pl.BlockSpec((1, C), lambda i: (0, 0)),
        ],
        out_specs=pl.BlockSpec((1, C), lambda i: (0, 0)),
        out_shape=jax.ShapeDtypeStruct((1, C), jnp.float32),
        scratch_shapes=[pltpu.VMEM((1, H), jnp.float32)],
    )(part, norms, W, b.reshape(1, H), Wc, bc.reshape(1, C))


def kernel(h, edge_index, W1, b1, W2, b2, Wc, bc):
    edge_flat = edge_index.astype(jnp.int32).reshape(2 * E)
    deg_part = _deg_call(edge_flat)
    norms, y1 = _norms_call(deg_part, h)
    part1 = _agg_call(y1, edge_flat)
    y2 = _layer_call(part1, norms, W1, b1)
    part2 = _agg_call(y2, edge_flat)
    return _final_call(part2, norms, W2, b2, Wc, bc)


# speedup vs baseline: 1.0494x; 1.0250x over previous
import jax
import jax.numpy as jnp
from jax import lax
from jax.experimental import pallas as pl
from jax.experimental.pallas import tpu as pltpu, tpu_sc as plsc

N = 10000
E = 320000
D = 128
H = 128
C = 10

NC = 2
NS = 16
NW = NC * NS
EPW = E // NW
CH = 80
NCH = EPW // CH
NP = 10240
RPT = NP // NS
R = 5120
G = NP // R


def _sc_mesh():
    return plsc.VectorSubcoreMesh(core_axis_name="c", subcore_axis_name="s")


def _deg_body(edge_hbm, out_hbm, src_v, dst_v, dego_v, degi_v, sem):
    c = lax.axis_index("c")
    s = lax.axis_index("s")
    wid = c * NS + s
    base = pl.multiple_of(wid * EPW, 8)
    zeros16 = jnp.zeros((16,), jnp.float32)
    ones16 = jnp.ones((16,), jnp.float32)

    pltpu.async_copy(edge_hbm.at[pl.ds(base, EPW)], src_v, sem.at[0])
    pltpu.async_copy(edge_hbm.at[pl.ds(E + base, EPW)], dst_v, sem.at[1])

    def zbody(i, carry):
        for j in range(5):
            dego_v[pl.ds((i * 5 + j) * 16, 16)] = zeros16
            degi_v[pl.ds((i * 5 + j) * 16, 16)] = zeros16
        return carry

    lax.fori_loop(0, N // 80, zbody, 0)

    pltpu.make_async_copy(edge_hbm.at[pl.ds(base, EPW)], src_v,
                          sem.at[0]).wait()
    pltpu.make_async_copy(edge_hbm.at[pl.ds(E + base, EPW)], dst_v,
                          sem.at[1]).wait()

    def abody(i, carry):
        for j in range(5):
            off = (i * 5 + j) * 16
            sidx = src_v[pl.ds(off, 16)]
            didx = dst_v[pl.ds(off, 16)]
            plsc.addupdate_scatter(dego_v, [sidx], ones16)
            plsc.addupdate_scatter(degi_v, [didx], ones16)
        return carry

    lax.fori_loop(0, EPW // 80, abody, 0)

    pltpu.sync_copy(dego_v, out_hbm.at[pl.ds(pl.multiple_of(wid * N, 8), N)])
    pltpu.sync_copy(
        degi_v, out_hbm.at[pl.ds(pl.multiple_of((NW + wid) * N, 8), N)])


def _deg_call(edge_flat):
    f = pl.kernel(
        _deg_body,
        out_type=jax.ShapeDtypeStruct((2 * NW * N,), jnp.float32),
        mesh=_sc_mesh(),
        compiler_params=pltpu.CompilerParams(needs_layout_passes=False),
        scratch_types=[
            pltpu.VMEM((EPW,), jnp.int32),
            pltpu.VMEM((EPW,), jnp.int32),
            pltpu.VMEM((N,), jnp.float32),
            pltpu.VMEM((N,), jnp.float32),
            pltpu.SemaphoreType.DMA((2,)),
        ],
    )
    return f(edge_flat)


def _agg_body(y_hbm, edge_hbm, out_hbm, src_all, dst_all, dstc_v, rows_v,
              acc_sh, sem, sem_s, sem_i):
    c = lax.axis_index("c")
    s = lax.axis_index("s")
    wid = c * NS + s
    base = pl.multiple_of(wid * EPW, 8)
    zeros16 = jnp.zeros((16,), jnp.float32)

    pltpu.async_copy(edge_hbm.at[pl.ds(base, EPW)], src_all, sem_i.at[0])
    pltpu.async_copy(edge_hbm.at[pl.ds(E + base, EPW)], dst_all, sem_i.at[1])

    def zbody(i, carry):
        for j in range(8):
            rows_v[1, i, pl.ds(j * 16, 16)] = zeros16
        return carry

    lax.fori_loop(0, CH, zbody, 0)
    pltpu.make_async_copy(edge_hbm.at[pl.ds(base, EPW)], src_all,
                          sem_i.at[0]).wait()

    def src_sl(k):
        return src_all.at[pl.ds(pl.multiple_of(k * CH, 8), CH)]

    def start_gather(k, b):
        pltpu.async_copy(y_hbm.at[src_sl(k)], rows_v.at[b], sem.at[b])

    def wait_gather(k, b):
        pltpu.make_async_copy(y_hbm.at[src_sl(k)], rows_v.at[b],
                              sem.at[b]).wait()

    def start_scatter(k, b):
        off = pl.multiple_of(k * CH, 8)
        for i in range(CH // 16):
            dstc_v[b, pl.ds(i * 16, 16)] = dst_all[pl.ds(off + i * 16, 16)]
        pltpu.async_copy(rows_v.at[b], acc_sh.at[dstc_v.at[b]], sem_s.at[b],
                         add=True)

    def wait_scatter(b):
        pltpu.make_async_copy(rows_v.at[b], acc_sh.at[dstc_v.at[b]],
                              sem_s.at[b]).wait()

    start_gather(0, 0)
    row0 = pl.multiple_of(s * RPT, 8)
    for r in range(RPT // CH):
        pltpu.sync_copy(rows_v.at[1], acc_sh.at[pl.ds(row0 + r * CH, CH)])
    pltpu.make_async_copy(edge_hbm.at[pl.ds(E + base, EPW)], dst_all,
                          sem_i.at[1]).wait()
    plsc.subcore_barrier()

    def ebody(g, carry):
        k = g * 2
        wait_gather(k, 0)

        @pl.when(k >= 2)
        def _():
            wait_scatter(1)

        start_gather(k + 1, 1)
        start_scatter(k, 0)
        wait_gather(k + 1, 1)
        wait_scatter(0)

        @pl.when(k + 2 < NCH)
        def _():
            start_gather(k + 2, 0)

        start_scatter(k + 1, 1)
        return carry

    lax.fori_loop(0, NCH // 2, ebody, 0)
    wait_scatter(1)
    wait_gather(NCH - 1, 0)
    start_scatter(NCH - 1, 0)
    wait_scatter(0)
    plsc.subcore_barrier()
    pltpu.sync_copy(
        acc_sh.at[pl.ds(row0, RPT)],
        out_hbm.at[pl.ds(pl.multiple_of(c * NP + s * RPT, 8), RPT)])


def _agg_call(y, edge_flat):
    f = pl.kernel(
        _agg_body,
        out_type=jax.ShapeDtypeStruct((NC * NP, D), jnp.float32),
        mesh=_sc_mesh(),
        compiler_params=pltpu.CompilerParams(needs_layout_passes=False),
        scratch_types=[
            pltpu.VMEM((EPW,), jnp.int32),
            pltpu.VMEM((EPW,), jnp.int32),
            pltpu.VMEM((2, CH), jnp.int32),
            pltpu.VMEM((2, CH, D), jnp.float32),
            pltpu.VMEM_SHARED((NP, D), jnp.float32),
            pltpu.SemaphoreType.DMA((2,)),
            pltpu.SemaphoreType.DMA((2,)),
            pltpu.SemaphoreType.DMA((2,)),
        ],
    )
    return f(y, edge_flat).reshape(NC, NP, D)


def _norms_body(deg_ref, h_ref, norms_ref, y1_ref):
    dp = deg_ref[...]
    dego = jnp.sum(dp[:NW], axis=0)
    degi = jnp.sum(dp[NW:], axis=0)
    no = lax.rsqrt(jnp.maximum(dego, 1.0))
    ni = lax.rsqrt(jnp.maximum(degi, 1.0))
    norms_ref[...] = jnp.stack([no, ni])
    y1_ref[...] = h_ref[...] * no[:, None]


def _norms_call(deg_part, h):
    return pl.pallas_call(
        _norms_body,
        grid=(G,),
        in_specs=[
            pl.BlockSpec((2 * NW, R), lambda i: (0, i)),
            pl.BlockSpec((R, D), lambda i: (i, 0)),
        ],
        out_specs=[
            pl.BlockSpec((2, R), lambda i: (0, i)),
            pl.BlockSpec((R, D), lambda i: (i, 0)),
        ],
        out_shape=[
            jax.ShapeDtypeStruct((2, N), jnp.float32),
            jax.ShapeDtypeStruct((N, D), jnp.float32),
        ],
    )(deg_part.reshape(2 * NW, N), h)


def _layer_body(part_ref, norms_ref, w_ref, b_ref, y2_ref):
    a = part_ref[0] + part_ref[1]
    nr = norms_ref[...]
    a = a * nr[1][:, None]
    z = jnp.dot(a, w_ref[...], preferred_element_type=jnp.float32) + b_ref[...]
    z = jnp.maximum(z, 0.0)
    y2_ref[...] = z * nr[0][:, None]


def _layer_call(part, norms, W, b):
    return pl.pallas_call(
        _layer_body,
        grid=(G,),
        in_specs=[
            pl.BlockSpec((NC, R, D), lambda i: (0, i, 0)),
            pl.BlockSpec((2, R), lambda i: (0, i)),
            pl.BlockSpec((D, H), lambda i: (0, 0)),
            pl.BlockSpec((1, H), lambda i: (0, 0)),
        ],
        out_specs=pl.BlockSpec((R, H), lambda i: (i, 0)),
        out_shape=jax.ShapeDtypeStruct((N, H), jnp.float32),
    )(part, norms, W, b.reshape(1, H))


def _final_body(part_ref, norms_ref, w_ref, b_ref, wc_ref, bc_ref, out_ref,
                acc_ref):
    i = pl.program_id(0)
    a = part_ref[0] + part_ref[1]
    nr = norms_ref[...]
    a = a * nr[1][:, None]
    z = jnp.dot(a, w_ref[...], preferred_element_type=jnp.float32) + b_ref[...]
    z = jnp.maximum(z, 0.0)
    rows = lax.broadcasted_iota(jnp.int32, z.shape, 0) + i * R
    z = jnp.where(rows < N, z, 0.0)

    @pl.when(i == 0)
    def _():
        acc_ref[...] = jnp.zeros_like(acc_ref)

    acc_ref[...] += jnp.sum(z, axis=0, keepdims=True)
    out_ref[...] = (
        jnp.dot(acc_ref[...] * (1.0 / N), wc_ref[...],
                preferred_element_type=jnp.float32) + bc_ref[...]
    )


def _final_call(part, norms, W, b, Wc, bc):
    return pl.pallas_call(
        _final_body,
        grid=(G,),
        in_specs=[
            pl.BlockSpec((NC, R, D), lambda i: (0, i, 0)),
            pl.BlockSpec((2, R), lambda i: (0, i)),
            pl.BlockSpec((H, H), lambda i: (0, 0)),
            pl.BlockSpec((1, H), lambda i: (0, 0)),
            pl.BlockSpec((H, C), lambda i: (0, 0)),
            pl.BlockSpec((1, C), lambda i: (0, 0)),
        ],
        out_specs=pl.BlockSpec((1, C), lambda i: (0, 0)),
        out_shape=jax.ShapeDtypeStruct((1, C), jnp.float32),
        scratch_shapes=[pltpu.VMEM((1, H), jnp.float32)],
    )(part, norms, W, b.reshape(1, H), Wc, bc.reshape(1, C))


def kernel(h, edge_index, W1, b1, W2, b2, Wc, bc):
    edge_flat = edge_index.astype(jnp.int32).reshape(2 * E)
    deg_part = _deg_call(edge_flat)
    norms, y1 = _norms_call(deg_part, h)
    part1 = _agg_call(y1, edge_flat)
    y2 = _layer_call(part1, norms, W1, b1)
    part2 = _agg_call(y2, edge_flat)
    return _final_call(part2, norms, W2, b2, Wc, bc)
